# baseline (speedup 1.0000x reference)
import jax
import jax.numpy as jnp
from jax.experimental import pallas as pl
from jax.experimental.pallas import tpu as pltpu

_N = 4096
_D_IN = 256
_D_OUT = 256
_TM = 256
_NBUF = 2
_NT = _N // _TM


def _graph_conv_kernel(a_hbm, x_hbm, wt_ref, b_ref, o_hbm,
                       a_buf, x_buf, y_ref, o_buf, a_sems, x_sem, o_sem):
    def a_copy(i, slot):
        return pltpu.make_async_copy(
            a_hbm.at[pl.ds(i * _TM, _TM), :], a_buf.at[slot], a_sems.at[slot])

    def o_copy(i):
        return pltpu.make_async_copy(
            o_buf.at[pl.ds(i * _TM, _TM), :],
            o_hbm.at[pl.ds(i * _TM, _TM), :], o_sem)

    x_dma = pltpu.make_async_copy(x_hbm, x_buf, x_sem)
    x_dma.start()
    for j in range(_NBUF):
        a_copy(j, j).start()
    x_dma.wait()
    y_ref[...] = jnp.dot(
        x_buf[...].astype(jnp.bfloat16),
        wt_ref[...].astype(jnp.bfloat16),
        preferred_element_type=jnp.float32,
    ).astype(jnp.bfloat16)

    for i in range(_NT):
        slot = i % _NBUF
        a_copy(i, slot).wait()
        acc = jnp.dot(
            a_buf[slot].astype(jnp.bfloat16),
            y_ref[...],
            preferred_element_type=jnp.float32,
        ) + b_ref[...]
        o_buf[pl.ds(i * _TM, _TM), :] = acc
        if i + _NBUF < _NT:
            a_copy(i + _NBUF, slot).start()
        o_copy(i).start()

    for i in range(_NT):
        o_copy(i).wait()


def kernel(A, x, W, b):
    wt = W.T
    b2 = b.reshape(1, _D_OUT)
    return pl.pallas_call(
        _graph_conv_kernel,
        in_specs=[
            pl.BlockSpec(memory_space=pltpu.MemorySpace.HBM),
            pl.BlockSpec(memory_space=pltpu.MemorySpace.HBM),
            pl.BlockSpec(memory_space=pltpu.MemorySpace.VMEM),
            pl.BlockSpec(memory_space=pltpu.MemorySpace.VMEM),
        ],
        out_specs=pl.BlockSpec(memory_space=pltpu.MemorySpace.HBM),
        out_shape=jax.ShapeDtypeStruct((_N, _D_OUT), jnp.float32),
        scratch_shapes=[
            pltpu.VMEM((_NBUF, _TM, _N), jnp.float32),
            pltpu.VMEM((_N, _D_IN), jnp.float32),
            pltpu.VMEM((_N, _D_OUT), jnp.bfloat16),
            pltpu.VMEM((_N, _D_OUT), jnp.float32),
            pltpu.SemaphoreType.DMA((_NBUF,)),
            pltpu.SemaphoreType.DMA,
            pltpu.SemaphoreType.DMA,
        ],
    )(A, x, wt, b2)


# speedup vs baseline: 1.1490x; 1.1490x over previous
import jax
import jax.numpy as jnp
from jax.experimental import pallas as pl
from jax.experimental.pallas import tpu as pltpu

_N = 4096
_D_IN = 256
_D_OUT = 256
_TM = 256
_NBUF = 4
_NT = _N // _TM


def _graph_conv_kernel(a_hbm, x_hbm, wt_ref, b_ref, o_hbm,
                       a_buf, x_buf, y_ref, o_buf, a_sems, x_sem, o_sem):
    def a_copy(i, slot):
        return pltpu.make_async_copy(
            a_hbm.at[pl.ds(i * _TM, _TM), :], a_buf.at[slot], a_sems.at[slot])

    def o_copy(i):
        return pltpu.make_async_copy(
            o_buf.at[pl.ds(i * _TM, _TM), :],
            o_hbm.at[pl.ds(i * _TM, _TM), :], o_sem)

    x_dma = pltpu.make_async_copy(x_hbm, x_buf, x_sem)
    x_dma.start()
    for j in range(_NBUF):
        a_copy(j, j).start()
    x_dma.wait()
    y_ref[...] = jnp.dot(
        x_buf[...].astype(jnp.bfloat16),
        wt_ref[...].astype(jnp.bfloat16),
        preferred_element_type=jnp.float32,
    ).astype(jnp.bfloat16)

    for i in range(_NT):
        slot = i % _NBUF
        a_copy(i, slot).wait()
        acc = jnp.dot(
            a_buf[slot].astype(jnp.bfloat16),
            y_ref[...],
            preferred_element_type=jnp.float32,
        ) + b_ref[...]
        o_buf[pl.ds(i * _TM, _TM), :] = acc
        if i + _NBUF < _NT:
            a_copy(i + _NBUF, slot).start()
        o_copy(i).start()

    for i in range(_NT):
        o_copy(i).wait()


def kernel(A, x, W, b):
    wt = W.T
    b2 = b.reshape(1, _D_OUT)
    return pl.pallas_call(
        _graph_conv_kernel,
        in_specs=[
            pl.BlockSpec(memory_space=pltpu.MemorySpace.HBM),
            pl.BlockSpec(memory_space=pltpu.MemorySpace.HBM),
            pl.BlockSpec(memory_space=pltpu.MemorySpace.VMEM),
            pl.BlockSpec(memory_space=pltpu.MemorySpace.VMEM),
        ],
        out_specs=pl.BlockSpec(memory_space=pltpu.MemorySpace.HBM),
        out_shape=jax.ShapeDtypeStruct((_N, _D_OUT), jnp.float32),
        scratch_shapes=[
            pltpu.VMEM((_NBUF, _TM, _N), jnp.float32),
            pltpu.VMEM((_N, _D_IN), jnp.float32),
            pltpu.VMEM((_N, _D_OUT), jnp.bfloat16),
            pltpu.VMEM((_N, _D_OUT), jnp.float32),
            pltpu.SemaphoreType.DMA((_NBUF,)),
            pltpu.SemaphoreType.DMA,
            pltpu.SemaphoreType.DMA,
        ],
    )(A, x, wt, b2)
